# baseline (speedup 1.0000x reference)
import functools

import jax
import jax.numpy as jnp
from jax.experimental import pallas as pl

N = 4096
F = 16
BM = 512


def _encode_kernel(adj_ref, w_ref, z_ref):
    z_ref[...] = jnp.dot(adj_ref[...], w_ref[...],
                         preferred_element_type=jnp.float32)


def _decode_kernel(zi_ref, z_ref, out_ref):
    out_ref[...] = jax.lax.dot_general(
        zi_ref[...], z_ref[...],
        dimension_numbers=(((1,), (1,)), ((), ())),
        preferred_element_type=jnp.float32)


@jax.jit
def kernel(adj, W):
    nb = N // BM
    z = pl.pallas_call(
        _encode_kernel,
        grid=(nb,),
        in_specs=[
            pl.BlockSpec((BM, N), lambda i: (i, 0)),
            pl.BlockSpec((N, F), lambda i: (0, 0)),
        ],
        out_specs=pl.BlockSpec((BM, F), lambda i: (i, 0)),
        out_shape=jax.ShapeDtypeStruct((N, F), jnp.float32),
    )(adj, W)

    out = pl.pallas_call(
        _decode_kernel,
        grid=(nb,),
        in_specs=[
            pl.BlockSpec((BM, F), lambda i: (i, 0)),
            pl.BlockSpec((N, F), lambda i: (0, 0)),
        ],
        out_specs=pl.BlockSpec((BM, N), lambda i: (i, 0)),
        out_shape=jax.ShapeDtypeStruct((N, N), jnp.float32),
    )(z, z)
    return out


# speedup vs baseline: 1.0164x; 1.0164x over previous
import functools

import jax
import jax.numpy as jnp
from jax.experimental import pallas as pl

N = 4096
F = 16
BM = 512


def _encode_kernel(adj_ref, w_ref, z_ref, zt_ref):
    z = jnp.dot(adj_ref[...], w_ref[...], preferred_element_type=jnp.float32)
    z_ref[...] = z
    zt_ref[...] = z.T


def _decode_kernel(zi_ref, zt_ref, out_ref):
    out_ref[...] = jnp.dot(zi_ref[...], zt_ref[...],
                           preferred_element_type=jnp.float32)


@jax.jit
def kernel(adj, W):
    nb = N // BM
    z, zt = pl.pallas_call(
        _encode_kernel,
        grid=(nb,),
        in_specs=[
            pl.BlockSpec((BM, N), lambda i: (i, 0)),
            pl.BlockSpec((N, F), lambda i: (0, 0)),
        ],
        out_specs=[
            pl.BlockSpec((BM, F), lambda i: (i, 0)),
            pl.BlockSpec((F, BM), lambda i: (0, i)),
        ],
        out_shape=[
            jax.ShapeDtypeStruct((N, F), jnp.float32),
            jax.ShapeDtypeStruct((F, N), jnp.float32),
        ],
    )(adj, W)

    out = pl.pallas_call(
        _decode_kernel,
        grid=(nb,),
        in_specs=[
            pl.BlockSpec((BM, F), lambda i: (i, 0)),
            pl.BlockSpec((F, N), lambda i: (0, 0)),
        ],
        out_specs=pl.BlockSpec((BM, N), lambda i: (i, 0)),
        out_shape=jax.ShapeDtypeStruct((N, N), jnp.float32),
    )(z, zt)
    return out


# speedup vs baseline: 1.0733x; 1.0560x over previous
import jax
import jax.numpy as jnp
from jax.experimental import pallas as pl
from jax.experimental.pallas import tpu as pltpu

N = 4096
F = 16
BM = 512
NB = N // BM


def _fused_kernel(adj_ref, w_ref, out_ref, z_scr, zt_scr):
    p = pl.program_id(0)
    i = pl.program_id(1)

    @pl.when(p == 0)
    def _encode():
        zi = jnp.dot(adj_ref[...], w_ref[...],
                     preferred_element_type=jnp.float32)
        z_scr[pl.ds(i * BM, BM), :] = zi
        zt_scr[:, pl.ds(i * BM, BM)] = zi.T

    @pl.when(p == 1)
    def _decode():
        out_ref[...] = jnp.dot(z_scr[pl.ds(i * BM, BM), :], zt_scr[...],
                               preferred_element_type=jnp.float32)


@jax.jit
def kernel(adj, W):
    out = pl.pallas_call(
        _fused_kernel,
        grid=(2, NB),
        in_specs=[
            pl.BlockSpec((BM, N), lambda p, i: (jnp.where(p == 0, i, NB - 1), 0)),
            pl.BlockSpec((N, F), lambda p, i: (0, 0)),
        ],
        out_specs=pl.BlockSpec((BM, N), lambda p, i: (jnp.where(p == 0, 0, i), 0)),
        out_shape=jax.ShapeDtypeStruct((N, N), jnp.float32),
        scratch_shapes=[
            pltpu.VMEM((N, F), jnp.float32),
            pltpu.VMEM((F, N), jnp.float32),
        ],
    )(adj, W)
    return out
